# Initial kernel scaffold; baseline (speedup 1.0000x reference)
import jax
import jax.numpy as jnp
from jax.experimental import pallas as pl


def kernel(x, edge_index, edge_weight, batch, W1, b1, g1, be1, W2, b2, g2, be2, W3, b3, g3, be3):
    raise NotImplementedError("write your pallas kernel here")


# speedup vs baseline: 13.3044x; 13.3044x over previous
import functools

import jax
import jax.numpy as jnp
from jax import lax
from jax.experimental import pallas as pl
from jax.experimental.pallas import tpu as pltpu
from jax.experimental.pallas import tpu_sc as plsc

N = 10000
E = 320000
F_IN = 128
DIM = 64
NUM_GRAPHS = 64

NPAD = 10240
NW = 32
CHUNK = 128
NCHUNK = 79
EPW = NCHUNK * CHUNK
RPT = NPAD // 16

_F32 = jnp.float32


def _mesh():
    return plsc.VectorSubcoreMesh(core_axis_name="c", subcore_axis_name="s")


@functools.partial(
    pl.kernel,
    mesh=_mesh(),
    compiler_params=pltpu.CompilerParams(needs_layout_passes=False, use_tc_tiling_on_sc=False),
    out_type=jax.ShapeDtypeStruct((2, NPAD), _F32),
    scratch_types=[
        pltpu.VMEM((NCHUNK, CHUNK), jnp.int32),
        pltpu.VMEM((NCHUNK, CHUNK), _F32),
        pltpu.VMEM((NPAD,), _F32),
        pltpu.VMEM_SHARED((16, NPAD), _F32),
        pltpu.VMEM((16, RPT), _F32),
        pltpu.VMEM((RPT,), _F32),
    ],
)
def _deg_kernel(dst_hbm, w_hbm, out_hbm, dstv, wv, priv, shared, buf, obuf):
    cid = lax.axis_index("c")
    sid = lax.axis_index("s")
    wid = cid * 16 + sid
    pltpu.sync_copy(dst_hbm.at[wid], dstv)
    pltpu.sync_copy(w_hbm.at[wid], wv)
    z16 = jnp.zeros((16,), _F32)

    def zero_body(i, carry):
        priv[pl.ds(i * 16, 16)] = z16
        return carry

    lax.fori_loop(0, NPAD // 16, zero_body, 0)

    def scat_body(r, carry):
        for c in range(CHUNK // 16):
            idx = dstv[r, pl.ds(c * 16, 16)]
            vals = wv[r, pl.ds(c * 16, 16)]
            plsc.addupdate_scatter(priv, [idx], vals)
        return carry

    lax.fori_loop(0, NCHUNK, scat_body, 0)
    pltpu.sync_copy(priv, shared.at[sid])
    plsc.subcore_barrier()
    pltpu.sync_copy(shared.at[:, pl.ds(sid * RPT, RPT)], buf)

    def red_body(k, carry):
        a = buf[0, pl.ds(k * 16, 16)]
        for t in range(1, 16):
            a = a + buf[t, pl.ds(k * 16, 16)]
        obuf[pl.ds(k * 16, 16)] = a
        return carry

    lax.fori_loop(0, RPT // 16, red_body, 0)
    pltpu.sync_copy(obuf, out_hbm.at[cid, pl.ds(sid * RPT, RPT)])


def _make_spmm(D):
    @functools.partial(
        pl.kernel,
        mesh=_mesh(),
        compiler_params=pltpu.CompilerParams(needs_layout_passes=False, use_tc_tiling_on_sc=False),
        out_type=jax.ShapeDtypeStruct((2, NPAD, D), _F32),
        scratch_types=[
            pltpu.VMEM((NCHUNK, CHUNK), jnp.int32),
            pltpu.VMEM((NCHUNK, CHUNK), jnp.int32),
            pltpu.VMEM((NCHUNK, CHUNK), _F32),
            pltpu.VMEM((CHUNK, D), _F32),
            pltpu.VMEM((RPT, D), _F32),
            pltpu.VMEM_SHARED((NPAD, D), _F32),
            pltpu.SemaphoreType.DMA,
        ],
    )
    def spmm(src_hbm, dst_hbm, w_hbm, hp_hbm, out_hbm,
             srcv, dstv, wv, rows, zbuf, acc, sem):
        cid = lax.axis_index("c")
        sid = lax.axis_index("s")
        wid = cid * 16 + sid
        z16 = jnp.zeros((16,), _F32)

        def zero_body(r, carry):
            for c in range(D // 16):
                zbuf[r, pl.ds(c * 16, 16)] = z16
            return carry

        lax.fori_loop(0, RPT, zero_body, 0)
        pltpu.sync_copy(zbuf, acc.at[pl.ds(sid * RPT, RPT)])
        pltpu.sync_copy(src_hbm.at[wid], srcv)
        pltpu.sync_copy(dst_hbm.at[wid], dstv)
        pltpu.sync_copy(w_hbm.at[wid], wv)
        plsc.subcore_barrier()

        def chunk_body(j, carry):
            pltpu.async_copy(hp_hbm.at[srcv.at[j]], rows, sem).wait()

            def scale_body(g, c2):
                wvec = wv[j, pl.ds(g * 16, 16)]
                for l in range(16):
                    e = g * 16 + l
                    wsc = wvec[l]
                    for c in range(D // 16):
                        rows[e, pl.ds(c * 16, 16)] = (
                            rows[e, pl.ds(c * 16, 16)] * wsc)
                return c2

            lax.fori_loop(0, CHUNK // 16, scale_body, 0)
            pltpu.sync_copy(rows, acc.at[dstv.at[j]], add=True)
            return carry

        lax.fori_loop(0, NCHUNK, chunk_body, 0)
        plsc.subcore_barrier()
        pltpu.sync_copy(acc.at[pl.ds(sid * RPT, RPT)], zbuf)
        pltpu.sync_copy(zbuf, out_hbm.at[cid, pl.ds(sid * RPT, RPT)])

    return spmm


_spmm64 = _make_spmm(DIM)
_spmm16 = _make_spmm(16)


def _tc1_body(x_ref, w1_ref, dp0_ref, dp1_ref, hp_ref, dinv_ref):
    deg = dp0_ref[...] + dp1_ref[...] + 1.0
    dinv = jnp.where(deg > 0, lax.rsqrt(jnp.maximum(deg, 1e-12)), 0.0)
    h = jnp.dot(x_ref[...], w1_ref[...], preferred_element_type=_F32)
    hp_ref[...] = h * dinv
    dinv_ref[...] = dinv


def _tc_mid_body(a0_ref, a1_ref, hp_ref, dinv_ref, b_ref, g_ref, be_ref,
                 wn_ref, out_ref, bcast):
    dinv = dinv_ref[...]
    o = dinv * (a0_ref[...] + a1_ref[...] + hp_ref[...]) + b_ref[...]
    m = jnp.mean(o, axis=0, keepdims=True)
    v = jnp.mean((o - m) ** 2, axis=0, keepdims=True)
    on = g_ref[...] * (o - m) * lax.rsqrt(v + 1e-5) + be_ref[...]
    r = jnp.maximum(on, 0.0)
    h = jnp.dot(r, wn_ref[...], preferred_element_type=_F32) * dinv
    if bcast:
        out_ref[...] = jnp.broadcast_to(h, out_ref.shape)
    else:
        out_ref[...] = h


def _tc4_body(a0_ref, a1_ref, hp_ref, dinv_ref, b_ref, g_ref, be_ref,
              batch_ref, out_ref):
    o = dinv_ref[...] * (a0_ref[...] + a1_ref[...] + hp_ref[...]) + b_ref[...]
    m = jnp.mean(o)
    v = jnp.mean((o - m) ** 2)
    on = g_ref[...] * (o - m) * lax.rsqrt(v + 1e-5) + be_ref[...]
    logit = on / 5.0
    ids = lax.broadcasted_iota(jnp.int32, (N, NUM_GRAPHS), 1)
    oh = batch_ref[...] == ids
    mg = jnp.max(jnp.where(oh, logit, -1e30), axis=0, keepdims=True)
    mb = jnp.sum(jnp.where(oh, mg, 0.0), axis=1, keepdims=True)
    z = jnp.exp(logit - mb)
    sg = jnp.sum(jnp.where(oh, z, 0.0), axis=0, keepdims=True)
    sb = jnp.sum(jnp.where(oh, sg, 0.0), axis=1, keepdims=True)
    out_ref[...] = z / (sb + 1e-16)


def _sds(shape):
    return jax.ShapeDtypeStruct(shape, _F32)


def kernel(x, edge_index, edge_weight, batch,
           W1, b1, g1, be1, W2, b2, g2, be2, W3, b3, g3, be3):
    src = edge_index[0]
    dst = edge_index[1]
    pad = NW * EPW - E
    srcp = jnp.concatenate([src, jnp.zeros((pad,), jnp.int32)]).reshape(
        NW, NCHUNK, CHUNK)
    dstp = jnp.concatenate([dst, jnp.zeros((pad,), jnp.int32)]).reshape(
        NW, NCHUNK, CHUNK)
    wp = jnp.concatenate([edge_weight, jnp.zeros((pad,), _F32)]).reshape(
        NW, NCHUNK, CHUNK)

    degp = _deg_kernel(dstp, wp)
    dp0 = degp[0, :N].reshape(N, 1)
    dp1 = degp[1, :N].reshape(N, 1)

    hp1, dinv = pl.pallas_call(
        _tc1_body, out_shape=[_sds((N, DIM)), _sds((N, 1))],
    )(x, W1, dp0, dp1)

    acc1 = _spmm64(srcp, dstp, wp, hp1)
    hp2 = pl.pallas_call(
        functools.partial(_tc_mid_body, bcast=False), out_shape=_sds((N, DIM)),
    )(acc1[0, :N], acc1[1, :N], hp1, dinv,
      b1.reshape(1, DIM), g1.reshape(1, DIM), be1.reshape(1, DIM), W2)

    acc2 = _spmm64(srcp, dstp, wp, hp2)
    hp3b = pl.pallas_call(
        functools.partial(_tc_mid_body, bcast=True), out_shape=_sds((N, 16)),
    )(acc2[0, :N], acc2[1, :N], hp2, dinv,
      b2.reshape(1, DIM), g2.reshape(1, DIM), be2.reshape(1, DIM), W3)

    acc3 = _spmm16(srcp, dstp, wp, hp3b)
    out = pl.pallas_call(
        _tc4_body, out_shape=_sds((N, 1)),
    )(acc3[0, :N, :1], acc3[1, :N, :1], hp3b[:, :1], dinv,
      b3.reshape(1, 1), g3.reshape(1, 1), be3.reshape(1, 1),
      batch.reshape(N, 1))
    return out


# speedup vs baseline: 15.0220x; 1.1291x over previous
import functools

import jax
import jax.numpy as jnp
from jax import lax
from jax.experimental import pallas as pl
from jax.experimental.pallas import tpu as pltpu
from jax.experimental.pallas import tpu_sc as plsc

N = 10000
E = 320000
F_IN = 128
DIM = 64
NUM_GRAPHS = 64

NPAD = 10240
NW = 32
CHUNK = 128
NB = 3
NCHUNK = 81
EPW = NCHUNK * CHUNK
RPT = NPAD // 16

_F32 = jnp.float32


def _mesh():
    return plsc.VectorSubcoreMesh(core_axis_name="c", subcore_axis_name="s")


@functools.partial(
    pl.kernel,
    mesh=_mesh(),
    compiler_params=pltpu.CompilerParams(needs_layout_passes=False, use_tc_tiling_on_sc=False),
    out_type=jax.ShapeDtypeStruct((2, NPAD), _F32),
    scratch_types=[
        pltpu.VMEM((NCHUNK, CHUNK), jnp.int32),
        pltpu.VMEM((NCHUNK, CHUNK), _F32),
        pltpu.VMEM((NPAD,), _F32),
        pltpu.VMEM_SHARED((16, NPAD), _F32),
        pltpu.VMEM((16, RPT), _F32),
        pltpu.VMEM((RPT,), _F32),
    ],
)
def _deg_kernel(dst_hbm, w_hbm, out_hbm, dstv, wv, priv, shared, buf, obuf):
    cid = lax.axis_index("c")
    sid = lax.axis_index("s")
    wid = cid * 16 + sid
    pltpu.sync_copy(dst_hbm.at[wid], dstv)
    pltpu.sync_copy(w_hbm.at[wid], wv)
    z16 = jnp.zeros((16,), _F32)

    def zero_body(i, carry):
        priv[pl.ds(i * 16, 16)] = z16
        return carry

    lax.fori_loop(0, NPAD // 16, zero_body, 0)

    def scat_body(r, carry):
        for c in range(CHUNK // 16):
            idx = dstv[r, pl.ds(c * 16, 16)]
            vals = wv[r, pl.ds(c * 16, 16)]
            plsc.addupdate_scatter(priv, [idx], vals)
        return carry

    lax.fori_loop(0, NCHUNK, scat_body, 0)
    pltpu.sync_copy(priv, shared.at[sid])
    plsc.subcore_barrier()
    pltpu.sync_copy(shared.at[:, pl.ds(sid * RPT, RPT)], buf)

    def red_body(k, carry):
        a = buf[0, pl.ds(k * 16, 16)]
        for t in range(1, 16):
            a = a + buf[t, pl.ds(k * 16, 16)]
        obuf[pl.ds(k * 16, 16)] = a
        return carry

    lax.fori_loop(0, RPT // 16, red_body, 0)
    pltpu.sync_copy(obuf, out_hbm.at[cid, pl.ds(sid * RPT, RPT)])


def _make_spmm(D):
    @functools.partial(
        pl.kernel,
        mesh=_mesh(),
        compiler_params=pltpu.CompilerParams(needs_layout_passes=False, use_tc_tiling_on_sc=False),
        out_type=jax.ShapeDtypeStruct((2, NPAD, D), _F32),
        scratch_types=[
            pltpu.VMEM((NCHUNK, CHUNK), jnp.int32),
            pltpu.VMEM((NCHUNK, CHUNK), jnp.int32),
            pltpu.VMEM((NCHUNK, CHUNK), _F32),
            [pltpu.VMEM((CHUNK, D), _F32)] * NB,
            [pltpu.VMEM((CHUNK, D), _F32)] * NB,
            pltpu.VMEM((CHUNK, D), _F32),
            pltpu.VMEM_SHARED((NPAD, D), _F32),
            [pltpu.SemaphoreType.DMA] * NB,
            [pltpu.SemaphoreType.DMA] * NB,
        ],
    )
    def spmm(src_hbm, dst_hbm, w_hbm, hp_hbm, out_hbm,
             srcv, dstv, wv, gbuf, sbuf, zbuf, acc, semg, sems):
        cid = lax.axis_index("c")
        sid = lax.axis_index("s")
        wid = cid * 16 + sid
        z16 = jnp.zeros((16,), _F32)

        def zero_body(r, carry):
            for c in range(D // 16):
                zbuf[r, pl.ds(c * 16, 16)] = z16
            return carry

        lax.fori_loop(0, CHUNK, zero_body, 0)
        for q in range(RPT // CHUNK):
            pltpu.sync_copy(zbuf, acc.at[pl.ds(sid * RPT + q * CHUNK, CHUNK)])
        pltpu.sync_copy(src_hbm.at[wid], srcv)
        pltpu.sync_copy(dst_hbm.at[wid], dstv)
        pltpu.sync_copy(w_hbm.at[wid], wv)
        plsc.subcore_barrier()

        for b in range(NB):
            pltpu.async_copy(hp_hbm.at[srcv.at[b]], gbuf[b], semg[b])

        def step(t, carry):
            for b in range(NB):
                j = t * NB + b
                pltpu.make_async_copy(
                    hp_hbm.at[srcv.at[j]], gbuf[b], semg[b]).wait()

                @pl.when(t > 0)
                def _wait_prev_scatter():
                    pltpu.make_async_copy(
                        sbuf[b], acc.at[dstv.at[j]], sems[b]).wait()

                def scale_body(g, c2):
                    wvec = wv[j, pl.ds(g * 16, 16)]
                    for l in range(16):
                        e = g * 16 + l
                        wsc = wvec[l]
                        for c in range(D // 16):
                            sbuf[b][e, pl.ds(c * 16, 16)] = (
                                gbuf[b][e, pl.ds(c * 16, 16)] * wsc)
                    return c2

                lax.fori_loop(0, CHUNK // 16, scale_body, 0)

                @pl.when(t < NCHUNK // NB - 1)
                def _issue_next_gather():
                    pltpu.async_copy(
                        hp_hbm.at[srcv.at[j + NB]], gbuf[b], semg[b])

                pltpu.async_copy(sbuf[b], acc.at[dstv.at[j]], sems[b],
                                 add=True)
            return carry

        lax.fori_loop(0, NCHUNK // NB, step, 0)
        for b in range(NB):
            pltpu.make_async_copy(
                sbuf[b], acc.at[dstv.at[NCHUNK - NB + b]], sems[b]).wait()
        plsc.subcore_barrier()
        for q in range(RPT // CHUNK):
            pltpu.sync_copy(
                acc.at[pl.ds(sid * RPT + q * CHUNK, CHUNK)], zbuf)
            pltpu.sync_copy(
                zbuf, out_hbm.at[cid, pl.ds(sid * RPT + q * CHUNK, CHUNK)])

    return spmm


_spmm64 = _make_spmm(DIM)
_spmm16 = _make_spmm(16)


def _tc1_body(x_ref, w1_ref, dp0_ref, dp1_ref, hp_ref, dinv_ref):
    deg = dp0_ref[...] + dp1_ref[...] + 1.0
    dinv = jnp.where(deg > 0, lax.rsqrt(jnp.maximum(deg, 1e-12)), 0.0)
    h = jnp.dot(x_ref[...], w1_ref[...], preferred_element_type=_F32)
    hp_ref[...] = h * dinv
    dinv_ref[...] = dinv


def _tc_mid_body(a0_ref, a1_ref, hp_ref, dinv_ref, b_ref, g_ref, be_ref,
                 wn_ref, out_ref, bcast):
    dinv = dinv_ref[...]
    o = dinv * (a0_ref[...] + a1_ref[...] + hp_ref[...]) + b_ref[...]
    m = jnp.mean(o, axis=0, keepdims=True)
    v = jnp.mean((o - m) ** 2, axis=0, keepdims=True)
    on = g_ref[...] * (o - m) * lax.rsqrt(v + 1e-5) + be_ref[...]
    r = jnp.maximum(on, 0.0)
    h = jnp.dot(r, wn_ref[...], preferred_element_type=_F32) * dinv
    if bcast:
        out_ref[...] = jnp.broadcast_to(h, out_ref.shape)
    else:
        out_ref[...] = h


def _tc4_body(a0_ref, a1_ref, hp_ref, dinv_ref, b_ref, g_ref, be_ref,
              batch_ref, out_ref):
    o = dinv_ref[...] * (a0_ref[...] + a1_ref[...] + hp_ref[...]) + b_ref[...]
    m = jnp.mean(o)
    v = jnp.mean((o - m) ** 2)
    on = g_ref[...] * (o - m) * lax.rsqrt(v + 1e-5) + be_ref[...]
    logit = on / 5.0
    ids = lax.broadcasted_iota(jnp.int32, (N, NUM_GRAPHS), 1)
    oh = batch_ref[...] == ids
    mg = jnp.max(jnp.where(oh, logit, -1e30), axis=0, keepdims=True)
    mb = jnp.sum(jnp.where(oh, mg, 0.0), axis=1, keepdims=True)
    z = jnp.exp(logit - mb)
    sg = jnp.sum(jnp.where(oh, z, 0.0), axis=0, keepdims=True)
    sb = jnp.sum(jnp.where(oh, sg, 0.0), axis=1, keepdims=True)
    out_ref[...] = z / (sb + 1e-16)


def _sds(shape):
    return jax.ShapeDtypeStruct(shape, _F32)


def kernel(x, edge_index, edge_weight, batch,
           W1, b1, g1, be1, W2, b2, g2, be2, W3, b3, g3, be3):
    src = edge_index[0]
    dst = edge_index[1]
    pad = NW * EPW - E
    srcp = jnp.concatenate([src, jnp.zeros((pad,), jnp.int32)]).reshape(
        NW, NCHUNK, CHUNK)
    dstp = jnp.concatenate([dst, jnp.zeros((pad,), jnp.int32)]).reshape(
        NW, NCHUNK, CHUNK)
    wp = jnp.concatenate([edge_weight, jnp.zeros((pad,), _F32)]).reshape(
        NW, NCHUNK, CHUNK)

    degp = _deg_kernel(dstp, wp)
    dp0 = degp[0, :N].reshape(N, 1)
    dp1 = degp[1, :N].reshape(N, 1)

    hp1, dinv = pl.pallas_call(
        _tc1_body, out_shape=[_sds((N, DIM)), _sds((N, 1))],
    )(x, W1, dp0, dp1)

    acc1 = _spmm64(srcp, dstp, wp, hp1)
    hp2 = pl.pallas_call(
        functools.partial(_tc_mid_body, bcast=False), out_shape=_sds((N, DIM)),
    )(acc1[0, :N], acc1[1, :N], hp1, dinv,
      b1.reshape(1, DIM), g1.reshape(1, DIM), be1.reshape(1, DIM), W2)

    acc2 = _spmm64(srcp, dstp, wp, hp2)
    hp3b = pl.pallas_call(
        functools.partial(_tc_mid_body, bcast=True), out_shape=_sds((N, 16)),
    )(acc2[0, :N], acc2[1, :N], hp2, dinv,
      b2.reshape(1, DIM), g2.reshape(1, DIM), be2.reshape(1, DIM), W3)

    acc3 = _spmm16(srcp, dstp, wp, hp3b)
    out = pl.pallas_call(
        _tc4_body, out_shape=_sds((N, 1)),
    )(acc3[0, :N, :1], acc3[1, :N, :1], hp3b[:, :1], dinv,
      b3.reshape(1, 1), g3.reshape(1, 1), be3.reshape(1, 1),
      batch.reshape(N, 1))
    return out


# speedup vs baseline: 15.1753x; 1.0102x over previous
import functools

import jax
import jax.numpy as jnp
from jax import lax
from jax.experimental import pallas as pl
from jax.experimental.pallas import tpu as pltpu
from jax.experimental.pallas import tpu_sc as plsc

N = 10000
E = 320000
F_IN = 128
DIM = 64
NUM_GRAPHS = 64

NPAD = 10240
NW = 32
CHUNK = 128
NB = 3
NCHUNK = 81
EPW = NCHUNK * CHUNK
RPT = NPAD // 16

_F32 = jnp.float32


def _mesh():
    return plsc.VectorSubcoreMesh(core_axis_name="c", subcore_axis_name="s")


@functools.partial(
    pl.kernel,
    mesh=_mesh(),
    compiler_params=pltpu.CompilerParams(needs_layout_passes=False, use_tc_tiling_on_sc=False),
    out_type=jax.ShapeDtypeStruct((2, NPAD), _F32),
    scratch_types=[
        pltpu.VMEM((NCHUNK, CHUNK), jnp.int32),
        pltpu.VMEM((NCHUNK, CHUNK), _F32),
        pltpu.VMEM((NPAD,), _F32),
        pltpu.VMEM_SHARED((16, NPAD), _F32),
        pltpu.VMEM((16, RPT), _F32),
        pltpu.VMEM((RPT,), _F32),
    ],
)
def _deg_kernel(dst_hbm, w_hbm, out_hbm, dstv, wv, priv, shared, buf, obuf):
    cid = lax.axis_index("c")
    sid = lax.axis_index("s")
    wid = cid * 16 + sid
    pltpu.sync_copy(dst_hbm.at[wid], dstv)
    pltpu.sync_copy(w_hbm.at[wid], wv)
    z16 = jnp.zeros((16,), _F32)

    def zero_body(i, carry):
        priv[pl.ds(i * 16, 16)] = z16
        return carry

    lax.fori_loop(0, NPAD // 16, zero_body, 0)

    def scat_body(r, carry):
        for c in range(CHUNK // 16):
            idx = dstv[r, pl.ds(c * 16, 16)]
            vals = wv[r, pl.ds(c * 16, 16)]
            plsc.addupdate_scatter(priv, [idx], vals)
        return carry

    lax.fori_loop(0, NCHUNK, scat_body, 0)
    pltpu.sync_copy(priv, shared.at[sid])
    plsc.subcore_barrier()
    pltpu.sync_copy(shared.at[:, pl.ds(sid * RPT, RPT)], buf)

    def red_body(k, carry):
        a = buf[0, pl.ds(k * 16, 16)]
        for t in range(1, 16):
            a = a + buf[t, pl.ds(k * 16, 16)]
        obuf[pl.ds(k * 16, 16)] = a
        return carry

    lax.fori_loop(0, RPT // 16, red_body, 0)
    pltpu.sync_copy(obuf, out_hbm.at[cid, pl.ds(sid * RPT, RPT)])


def _make_spmm(D):
    @functools.partial(
        pl.kernel,
        mesh=_mesh(),
        compiler_params=pltpu.CompilerParams(needs_layout_passes=False, use_tc_tiling_on_sc=False),
        out_type=jax.ShapeDtypeStruct((2, NPAD, D), _F32),
        scratch_types=[
            pltpu.VMEM((NCHUNK, CHUNK), jnp.int32),
            pltpu.VMEM((NCHUNK, CHUNK), jnp.int32),
            pltpu.VMEM((NCHUNK, CHUNK), _F32),
            [pltpu.VMEM((CHUNK, D), _F32)] * NB,
            [pltpu.VMEM((CHUNK, D), _F32)] * NB,
            pltpu.VMEM((CHUNK, D), _F32),
            pltpu.VMEM_SHARED((NPAD, D), _F32),
            [pltpu.SemaphoreType.DMA] * NB,
            [pltpu.SemaphoreType.DMA] * NB,
        ],
    )
    def spmm(src_hbm, dst_hbm, w_hbm, hp_hbm, out_hbm,
             srcv, dstv, wv, gbuf, sbuf, zbuf, acc, semg, sems):
        cid = lax.axis_index("c")
        sid = lax.axis_index("s")
        wid = cid * 16 + sid
        z16 = jnp.zeros((16,), _F32)

        def zero_body(r, carry):
            for c in range(D // 16):
                zbuf[r, pl.ds(c * 16, 16)] = z16
            return carry

        lax.fori_loop(0, CHUNK, zero_body, 0)
        for q in range(RPT // CHUNK):
            pltpu.sync_copy(zbuf, acc.at[pl.ds(sid * RPT + q * CHUNK, CHUNK)])
        pltpu.sync_copy(src_hbm.at[wid], srcv)
        pltpu.sync_copy(dst_hbm.at[wid], dstv)
        pltpu.sync_copy(w_hbm.at[wid], wv)
        plsc.subcore_barrier()

        for b in range(NB):
            pltpu.async_copy(hp_hbm.at[srcv.at[b]], gbuf[b], semg[b])

        def step(t, carry):
            for b in range(NB):
                j = t * NB + b
                pltpu.make_async_copy(
                    hp_hbm.at[srcv.at[j]], gbuf[b], semg[b]).wait()

                @pl.when(t > 0)
                def _wait_prev_scatter():
                    pltpu.make_async_copy(
                        sbuf[b], acc.at[dstv.at[j]], sems[b]).wait()

                def scale_body(g, c2):
                    wvec = wv[j, pl.ds(g * 16, 16)]
                    for l in range(16):
                        e = g * 16 + l
                        wsc = wvec[l]
                        for c in range(D // 16):
                            sbuf[b][e, pl.ds(c * 16, 16)] = (
                                gbuf[b][e, pl.ds(c * 16, 16)] * wsc)
                    return c2

                lax.fori_loop(0, CHUNK // 16, scale_body, 0)

                @pl.when(t < NCHUNK // NB - 1)
                def _issue_next_gather():
                    pltpu.async_copy(
                        hp_hbm.at[srcv.at[j + NB]], gbuf[b], semg[b])

                pltpu.async_copy(sbuf[b], acc.at[dstv.at[j]], sems[b],
                                 add=True)
            return carry

        lax.fori_loop(0, NCHUNK // NB, step, 0)
        for b in range(NB):
            pltpu.make_async_copy(
                sbuf[b], acc.at[dstv.at[NCHUNK - NB + b]], sems[b]).wait()
        plsc.subcore_barrier()
        for q in range(RPT // CHUNK):
            pltpu.sync_copy(
                acc.at[pl.ds(sid * RPT + q * CHUNK, CHUNK)], zbuf)
            pltpu.sync_copy(
                zbuf, out_hbm.at[cid, pl.ds(sid * RPT + q * CHUNK, CHUNK)])

    return spmm


_spmm64 = _make_spmm(DIM)
_spmm16 = _make_spmm(16)


def _tc1_body(x_ref, w1_ref, dp0_ref, dp1_ref, hp_ref, dinv_ref):
    deg = dp0_ref[...] + dp1_ref[...] + 1.0
    dinv = jnp.where(deg > 0, lax.rsqrt(jnp.maximum(deg, 1e-12)), 0.0)
    h = jnp.dot(x_ref[...], w1_ref[...], preferred_element_type=_F32)
    hp_ref[...] = h * dinv
    dinv_ref[...] = dinv


def _tc_mid_body(a0_ref, a1_ref, hp_ref, dinv_ref, b_ref, g_ref, be_ref,
                 wn_ref, out_ref, bcast):
    dinv = dinv_ref[...]
    o = dinv * (a0_ref[...] + a1_ref[...] + hp_ref[...]) + b_ref[...]
    m = jnp.mean(o, axis=0, keepdims=True)
    v = jnp.mean((o - m) ** 2, axis=0, keepdims=True)
    on = g_ref[...] * (o - m) * lax.rsqrt(v + 1e-5) + be_ref[...]
    r = jnp.maximum(on, 0.0)
    h = jnp.dot(r, wn_ref[...], preferred_element_type=_F32) * dinv
    if bcast:
        out_ref[...] = jnp.broadcast_to(h, out_ref.shape)
    else:
        out_ref[...] = h


def _tc4_body(a0_ref, a1_ref, hp_ref, dinv_ref, b_ref, g_ref, be_ref,
              batch_ref, out_ref):
    o = dinv_ref[...] * (a0_ref[...] + a1_ref[...] + hp_ref[...]) + b_ref[...]
    m = jnp.mean(o)
    v = jnp.mean((o - m) ** 2)
    on = g_ref[...] * (o - m) * lax.rsqrt(v + 1e-5) + be_ref[...]
    logit = on / 5.0
    ids = lax.broadcasted_iota(jnp.int32, (N, NUM_GRAPHS), 1)
    oh = batch_ref[...] == ids
    mg = jnp.max(jnp.where(oh, logit, -1e30), axis=0, keepdims=True)
    mb = jnp.sum(jnp.where(oh, mg, 0.0), axis=1, keepdims=True)
    z = jnp.exp(logit - mb)
    sg = jnp.sum(jnp.where(oh, z, 0.0), axis=0, keepdims=True)
    sb = jnp.sum(jnp.where(oh, sg, 0.0), axis=1, keepdims=True)
    out_ref[...] = z / (sb + 1e-16)


def _sds(shape):
    return jax.ShapeDtypeStruct(shape, _F32)


def kernel(x, edge_index, edge_weight, batch,
           W1, b1, g1, be1, W2, b2, g2, be2, W3, b3, g3, be3):
    src = edge_index[0]
    dst = edge_index[1]
    pad = NW * EPW - E
    pad_dst = (jnp.arange(pad, dtype=jnp.int32) % (NPAD - N)) + N
    srcp = jnp.concatenate([src, jnp.zeros((pad,), jnp.int32)]).reshape(
        NW, NCHUNK, CHUNK)
    dstp = jnp.concatenate([dst, pad_dst]).reshape(
        NW, NCHUNK, CHUNK)
    wp = jnp.concatenate([edge_weight, jnp.zeros((pad,), _F32)]).reshape(
        NW, NCHUNK, CHUNK)

    degp = _deg_kernel(dstp, wp)
    dp0 = degp[0, :N].reshape(N, 1)
    dp1 = degp[1, :N].reshape(N, 1)

    hp1, dinv = pl.pallas_call(
        _tc1_body, out_shape=[_sds((N, DIM)), _sds((N, 1))],
    )(x, W1, dp0, dp1)

    acc1 = _spmm64(srcp, dstp, wp, hp1)
    hp2 = pl.pallas_call(
        functools.partial(_tc_mid_body, bcast=False), out_shape=_sds((N, DIM)),
    )(acc1[0, :N], acc1[1, :N], hp1, dinv,
      b1.reshape(1, DIM), g1.reshape(1, DIM), be1.reshape(1, DIM), W2)

    acc2 = _spmm64(srcp, dstp, wp, hp2)
    hp3b = pl.pallas_call(
        functools.partial(_tc_mid_body, bcast=True), out_shape=_sds((N, 16)),
    )(acc2[0, :N], acc2[1, :N], hp2, dinv,
      b2.reshape(1, DIM), g2.reshape(1, DIM), be2.reshape(1, DIM), W3)

    acc3 = _spmm16(srcp, dstp, wp, hp3b)
    out = pl.pallas_call(
        _tc4_body, out_shape=_sds((N, 1)),
    )(acc3[0, :N, :1], acc3[1, :N, :1], hp3b[:, :1], dinv,
      b3.reshape(1, 1), g3.reshape(1, 1), be3.reshape(1, 1),
      batch.reshape(N, 1))
    return out
